# baseline (speedup 1.0000x reference)
import jax
import jax.numpy as jnp
from jax import lax
from jax.experimental import pallas as pl
from jax.experimental.pallas import tpu as pltpu
from jax.experimental.pallas import tpu_sc as plsc

EMBED_DIM = 128


def _sc_lookup(idx_hbm, table_hbm, out_hbm, idx_v, row_v, sem):
    wid = lax.axis_index("s") * 2 + lax.axis_index("c")

    @pl.when(wid == 0)
    def _():
        pltpu.sync_copy(idx_hbm, idx_v)
        pltpu.async_copy(table_hbm.at[idx_v], row_v, sem).wait()
        pltpu.sync_copy(row_v.at[0], out_hbm)


def kernel(weight, table):
    idx = jnp.asarray(weight, dtype=jnp.int32).reshape(1)
    mesh = plsc.VectorSubcoreMesh(core_axis_name="c", subcore_axis_name="s")
    out = pl.kernel(
        _sc_lookup,
        out_type=jax.ShapeDtypeStruct((EMBED_DIM,), jnp.float32),
        mesh=mesh,
        scratch_types=[
            pltpu.VMEM((1,), jnp.int32),
            pltpu.VMEM((1, EMBED_DIM), jnp.float32),
            pltpu.SemaphoreType.DMA,
        ],
    )(idx, table)
    return out


# speedup vs baseline: 1.0815x; 1.0815x over previous
import jax
import jax.numpy as jnp
from jax import lax
from jax.experimental import pallas as pl
from jax.experimental.pallas import tpu as pltpu
from jax.experimental.pallas import tpu_sc as plsc

EMBED_DIM = 128


def _scs_lookup(idx_hbm, table_hbm, out_hbm, idx_s):
    @pl.when(lax.axis_index("c") == 0)
    def _():
        pltpu.sync_copy(idx_hbm, idx_s)
        i = idx_s[0]
        pltpu.sync_copy(table_hbm.at[pl.ds(i, 1)], out_hbm)


def kernel(weight, table):
    idx = jnp.asarray(weight, dtype=jnp.int32).reshape(1)
    mesh = plsc.ScalarSubcoreMesh(axis_name="c", num_cores=2)
    out = pl.kernel(
        _scs_lookup,
        out_type=jax.ShapeDtypeStruct((1, EMBED_DIM), jnp.float32),
        mesh=mesh,
        scratch_types=[
            pltpu.SMEM((1,), jnp.int32),
        ],
    )(idx, table)
    return out[0]


# speedup vs baseline: 1.1783x; 1.0895x over previous
import jax
import jax.numpy as jnp
from jax import lax
from jax.experimental import pallas as pl
from jax.experimental.pallas import tpu as pltpu
from jax.experimental.pallas import tpu_sc as plsc

EMBED_DIM = 128


def _scs_lookup(idx_hbm, table_hbm, out_hbm, idx_s):
    pltpu.sync_copy(idx_hbm, idx_s)
    i = idx_s[0]
    pltpu.sync_copy(table_hbm.at[pl.ds(i, 1)], out_hbm)


def kernel(weight, table):
    idx = jnp.asarray(weight, dtype=jnp.int32).reshape(1)
    mesh = plsc.ScalarSubcoreMesh(axis_name="c", num_cores=1)
    out = pl.kernel(
        _scs_lookup,
        out_type=jax.ShapeDtypeStruct((1, EMBED_DIM), jnp.float32),
        mesh=mesh,
        scratch_types=[
            pltpu.SMEM((1,), jnp.int32),
        ],
    )(idx, table)
    return out[0]


# speedup vs baseline: 10.5936x; 8.9903x over previous
import jax
import jax.numpy as jnp
from jax.experimental import pallas as pl
from jax.experimental.pallas import tpu as pltpu

EMBED_DIM = 128


def _tc_body(idx_ref, block_ref, o_ref):
    r = idx_ref[0] % 8
    o_ref[...] = block_ref[pl.ds(r, 1), :]


def kernel(weight, table):
    idx = jnp.asarray(weight, dtype=jnp.int32).reshape(1)
    out = pl.pallas_call(
        _tc_body,
        grid_spec=pltpu.PrefetchScalarGridSpec(
            num_scalar_prefetch=1,
            grid=(1,),
            in_specs=[pl.BlockSpec((8, EMBED_DIM), lambda i, idx_ref: (idx_ref[0] // 8, 0))],
            out_specs=pl.BlockSpec((1, EMBED_DIM), lambda i, idx_ref: (0, 0)),
        ),
        out_shape=jax.ShapeDtypeStruct((1, EMBED_DIM), jnp.float32),
    )(idx, table)
    return out[0]
